# baseline (speedup 1.0000x reference)
import functools
import jax
import jax.numpy as jnp
from jax import lax
from jax.experimental import pallas as pl
from jax.experimental.pallas import tpu as pltpu, tpu_sc as plsc

DEPTH_ = 100000
BATCH_ = 1024

NC = 2
NS = 16
NW = NC * NS
ROWS_W = BATCH_ // NW

_mesh = plsc.VectorSubcoreMesh(core_axis_name="c", subcore_axis_name="s")


@functools.partial(
    pl.kernel,
    out_type=jax.ShapeDtypeStruct((BATCH_, DEPTH_), jnp.float32),
    mesh=_mesh,
    scratch_types=[
        pltpu.VMEM((1, DEPTH_), jnp.float32),
        pltpu.VMEM((ROWS_W,), jnp.int32),
        pltpu.SemaphoreType.DMA,
        pltpu.SemaphoreType.DMA,
    ],
)
def _sc_onehot(idx_hbm, zeros_hbm, eye_hbm, out_hbm, zrow, idx_v, zsem, psem):
    wid = lax.axis_index("s") * NC + lax.axis_index("c")
    base = wid * ROWS_W

    pltpu.sync_copy(zeros_hbm, zrow)
    pltpu.sync_copy(idx_hbm.at[pl.ds(base, ROWS_W)], idx_v)

    for r in range(ROWS_W):
        pltpu.make_async_copy(
            zrow, out_hbm.at[pl.ds(base + r, 1), :], zsem
        ).start()
    for r in range(ROWS_W):
        pltpu.make_async_copy(
            zrow, out_hbm.at[pl.ds(base + r, 1), :], zsem
        ).wait()

    for j in range(ROWS_W):
        idxg = idx_v[pl.ds((j // 16) * 16, 16)]
        c = idxg[j % 16]
        c0 = (c // 8) * 8
        pltpu.make_async_copy(
            eye_hbm.at[pl.ds(c - c0, 1), pl.ds(0, 8)],
            out_hbm.at[pl.ds(base + j, 1), pl.ds(c0, 8)],
            psem,
        ).start()
    for j in range(ROWS_W):
        idxg = idx_v[pl.ds((j // 16) * 16, 16)]
        c = idxg[j % 16]
        c0 = (c // 8) * 8
        pltpu.make_async_copy(
            eye_hbm.at[pl.ds(c - c0, 1), pl.ds(0, 8)],
            out_hbm.at[pl.ds(base + j, 1), pl.ds(c0, 8)],
            psem,
        ).wait()


def kernel(input):
    idx = input.astype(jnp.int32).reshape(BATCH_)
    zeros = jnp.zeros((1, DEPTH_), jnp.float32)
    eye = jnp.pad(jnp.eye(8, dtype=jnp.float32), ((0, 0), (0, DEPTH_ - 8)))
    return _sc_onehot(idx, zeros, eye)
